# initial kernel scaffold (unmeasured)
import functools

import jax
import jax.numpy as jnp
from jax import lax
from jax.experimental import pallas as pl
from jax.experimental.pallas import tpu as pltpu

N_DEV = 8


def kernel(x, w_mat, scale_x, scale_w):
    m_per, k = x.shape
    n_per = w_mat.shape[1]
    m_total = N_DEV * m_per

    def body(x_ref, w_ref, sx_ref, sw_ref, out_ref, comm_ref,
             send_sems, recv_sems):
        my = lax.axis_index("i")
        left = lax.rem(my - 1 + N_DEV, N_DEV)
        right = lax.rem(my + 1, N_DEV)

        barrier_sem = pltpu.get_barrier_semaphore()
        for nbr in (left, right):
            pl.semaphore_signal(
                barrier_sem, inc=1,
                device_id=(nbr,), device_id_type=pl.DeviceIdType.MESH,
            )
        pl.semaphore_wait(barrier_sem, 2)

        scale = sx_ref[0] * sw_ref[0]

        comm_ref[0] = x_ref[...]

        def gemm_slot(s):
            origin = lax.rem(my - s + N_DEV, N_DEV)
            acc = jnp.dot(comm_ref[s], w_ref[...],
                          preferred_element_type=jnp.float32)
            out_ref[pl.ds(origin * m_per, m_per), :] = acc * scale

        for h in range(N_DEV - 1):
            rdma = pltpu.make_async_remote_copy(
                src_ref=comm_ref.at[h],
                dst_ref=comm_ref.at[h + 1],
                send_sem=send_sems.at[h],
                recv_sem=recv_sems.at[h],
                device_id=(right,),
                device_id_type=pl.DeviceIdType.MESH,
            )
            rdma.start()
            gemm_slot(h)
            rdma.wait()
        gemm_slot(N_DEV - 1)

        @functools.partial(
            pl.run_scoped, second_barrier=pltpu.SemaphoreType.REGULAR
        )
        def _(second_barrier):
            for nbr in (left, right):
                pl.semaphore_signal(
                    second_barrier, inc=1,
                    device_id=(nbr,), device_id_type=pl.DeviceIdType.MESH,
                )
            pl.semaphore_wait(second_barrier, 2)

    return pl.pallas_call(
        body,
        out_shape=jax.ShapeDtypeStruct((m_total, n_per), jnp.float32),
        in_specs=[
            pl.BlockSpec(memory_space=pltpu.VMEM),
            pl.BlockSpec(memory_space=pltpu.VMEM),
            pl.BlockSpec(memory_space=pltpu.SMEM),
            pl.BlockSpec(memory_space=pltpu.SMEM),
        ],
        out_specs=pl.BlockSpec(memory_space=pltpu.VMEM),
        scratch_shapes=[
            pltpu.VMEM((N_DEV, m_per, k), x.dtype),
            pltpu.SemaphoreType.DMA((N_DEV - 1,)),
            pltpu.SemaphoreType.DMA((N_DEV - 1,)),
        ],
        compiler_params=pltpu.CompilerParams(collective_id=0),
    )(x, w_mat, scale_x, scale_w)


# baseline (device time: 185444 ns/iter reference)
import functools

import jax
import jax.numpy as jnp
from jax import lax
from jax.experimental import pallas as pl
from jax.experimental.pallas import tpu as pltpu

N_DEV = 8


def kernel(x, w_mat, scale_x, scale_w):
    m_per, k = x.shape
    n_per = w_mat.shape[1]
    m_total = N_DEV * m_per

    def body(x_ref, w_ref, sx_ref, sw_ref, out_ref, comm_ref, w_bf_ref,
             send_sems, recv_sems):
        my = lax.axis_index("i")
        left = lax.rem(my - 1 + N_DEV, N_DEV)
        right = lax.rem(my + 1, N_DEV)

        barrier_sem = pltpu.get_barrier_semaphore()
        for nbr in (left, right):
            pl.semaphore_signal(
                barrier_sem, inc=1,
                device_id=(nbr,), device_id_type=pl.DeviceIdType.MESH,
            )
        pl.semaphore_wait(barrier_sem, 2)

        scale = sx_ref[0] * sw_ref[0]

        comm_ref[0] = x_ref[...].astype(jnp.float8_e5m2)
        w_bf_ref[...] = w_ref[...].astype(jnp.bfloat16)

        def gemm_slot(s):
            origin = lax.rem(my - s + N_DEV, N_DEV)
            acc = jnp.dot(comm_ref[s].astype(jnp.bfloat16), w_bf_ref[...],
                          preferred_element_type=jnp.float32)
            out_ref[pl.ds(origin * m_per, m_per), :] = acc * scale

        for h in range(N_DEV - 1):
            rdma = pltpu.make_async_remote_copy(
                src_ref=comm_ref.at[h],
                dst_ref=comm_ref.at[h + 1],
                send_sem=send_sems.at[h],
                recv_sem=recv_sems.at[h],
                device_id=(right,),
                device_id_type=pl.DeviceIdType.MESH,
            )
            rdma.start()
            gemm_slot(h)
            rdma.wait()
        gemm_slot(N_DEV - 1)

        @functools.partial(
            pl.run_scoped, second_barrier=pltpu.SemaphoreType.REGULAR
        )
        def _(second_barrier):
            for nbr in (left, right):
                pl.semaphore_signal(
                    second_barrier, inc=1,
                    device_id=(nbr,), device_id_type=pl.DeviceIdType.MESH,
                )
            pl.semaphore_wait(second_barrier, 2)

    return pl.pallas_call(
        body,
        out_shape=jax.ShapeDtypeStruct((m_total, n_per), jnp.float32),
        in_specs=[
            pl.BlockSpec(memory_space=pltpu.VMEM),
            pl.BlockSpec(memory_space=pltpu.VMEM),
            pl.BlockSpec(memory_space=pltpu.SMEM),
            pl.BlockSpec(memory_space=pltpu.SMEM),
        ],
        out_specs=pl.BlockSpec(memory_space=pltpu.VMEM),
        scratch_shapes=[
            pltpu.VMEM((N_DEV, m_per, k), jnp.float8_e5m2),
            pltpu.VMEM((k, n_per), jnp.bfloat16),
            pltpu.SemaphoreType.DMA((N_DEV - 1,)),
            pltpu.SemaphoreType.DMA((N_DEV - 1,)),
        ],
        compiler_params=pltpu.CompilerParams(collective_id=0),
    )(x, w_mat, scale_x, scale_w)


# device time: 108697 ns/iter; 1.7061x vs baseline; 1.7061x over previous
import functools

import jax
import jax.numpy as jnp
from jax import lax
from jax.experimental import pallas as pl
from jax.experimental.pallas import tpu as pltpu

N_DEV = 8


def kernel(x, w_mat, scale_x, scale_w):
    m_per, k = x.shape
    n_per = w_mat.shape[1]
    m_total = N_DEV * m_per
    m_half = m_per // 2

    def body(x_ref, w_ref, sx_ref, sw_ref, out_ref, fwd_ref, bwd_ref,
             w_bf_ref, fs_sems, fr_sems, bs_sems, br_sems):
        my = lax.axis_index("i")
        left = lax.rem(my - 1 + N_DEV, N_DEV)
        right = lax.rem(my + 1, N_DEV)

        barrier_sem = pltpu.get_barrier_semaphore()
        for nbr in (left, right):
            pl.semaphore_signal(
                barrier_sem, inc=1,
                device_id=(nbr,), device_id_type=pl.DeviceIdType.MESH,
            )
        pl.semaphore_wait(barrier_sem, 2)

        scale = sx_ref[0] * sw_ref[0]

        fwd_ref[0] = x_ref[:m_half, :].astype(jnp.float8_e5m2)
        bwd_ref[0] = x_ref[m_half:, :].astype(jnp.float8_e5m2)
        w_bf_ref[...] = w_ref[...].astype(jnp.bfloat16)

        def gemm_slot(s):
            o_f = lax.rem(my - s + N_DEV, N_DEV)
            acc = jnp.dot(fwd_ref[s].astype(jnp.bfloat16), w_bf_ref[...],
                          preferred_element_type=jnp.float32)
            out_ref[pl.ds(o_f * m_per, m_half), :] = acc * scale
            o_b = lax.rem(my + s, N_DEV)
            acc = jnp.dot(bwd_ref[s].astype(jnp.bfloat16), w_bf_ref[...],
                          preferred_element_type=jnp.float32)
            out_ref[pl.ds(o_b * m_per + m_half, m_half), :] = acc * scale

        for h in range(N_DEV - 1):
            fwd = pltpu.make_async_remote_copy(
                src_ref=fwd_ref.at[h],
                dst_ref=fwd_ref.at[h + 1],
                send_sem=fs_sems.at[h],
                recv_sem=fr_sems.at[h],
                device_id=(right,),
                device_id_type=pl.DeviceIdType.MESH,
            )
            bwd = pltpu.make_async_remote_copy(
                src_ref=bwd_ref.at[h],
                dst_ref=bwd_ref.at[h + 1],
                send_sem=bs_sems.at[h],
                recv_sem=br_sems.at[h],
                device_id=(left,),
                device_id_type=pl.DeviceIdType.MESH,
            )
            fwd.start()
            bwd.start()
            gemm_slot(h)
            fwd.wait()
            bwd.wait()
        gemm_slot(N_DEV - 1)

        @functools.partial(
            pl.run_scoped, second_barrier=pltpu.SemaphoreType.REGULAR
        )
        def _(second_barrier):
            for nbr in (left, right):
                pl.semaphore_signal(
                    second_barrier, inc=1,
                    device_id=(nbr,), device_id_type=pl.DeviceIdType.MESH,
                )
            pl.semaphore_wait(second_barrier, 2)

    return pl.pallas_call(
        body,
        out_shape=jax.ShapeDtypeStruct((m_total, n_per), jnp.float32),
        in_specs=[
            pl.BlockSpec(memory_space=pltpu.VMEM),
            pl.BlockSpec(memory_space=pltpu.VMEM),
            pl.BlockSpec(memory_space=pltpu.SMEM),
            pl.BlockSpec(memory_space=pltpu.SMEM),
        ],
        out_specs=pl.BlockSpec(memory_space=pltpu.VMEM),
        scratch_shapes=[
            pltpu.VMEM((N_DEV, m_half, k), jnp.float8_e5m2),
            pltpu.VMEM((N_DEV, m_half, k), jnp.float8_e5m2),
            pltpu.VMEM((k, n_per), jnp.bfloat16),
            pltpu.SemaphoreType.DMA((N_DEV - 1,)),
            pltpu.SemaphoreType.DMA((N_DEV - 1,)),
            pltpu.SemaphoreType.DMA((N_DEV - 1,)),
            pltpu.SemaphoreType.DMA((N_DEV - 1,)),
        ],
        compiler_params=pltpu.CompilerParams(collective_id=0),
    )(x, w_mat, scale_x, scale_w)


# device time: 95948 ns/iter; 1.9328x vs baseline; 1.1329x over previous
import functools

import jax
import jax.numpy as jnp
from jax import lax
from jax.experimental import pallas as pl
from jax.experimental.pallas import tpu as pltpu

N_DEV = 8


def kernel(x, w_mat, scale_x, scale_w):
    m_per, k = x.shape
    n_per = w_mat.shape[1]
    m_total = N_DEV * m_per
    m_half = m_per // 2

    n_seg = 2
    seg = m_half // n_seg

    def body(x_ref, w_ref, sx_ref, sw_ref, out_ref, fwd_ref, bwd_ref,
             w_bf_ref, fs_sems, fr_sems, bs_sems, br_sems):
        my = lax.axis_index("i")
        left = lax.rem(my - 1 + N_DEV, N_DEV)
        right = lax.rem(my + 1, N_DEV)

        barrier_sem = pltpu.get_barrier_semaphore()
        for nbr in (left, right):
            pl.semaphore_signal(
                barrier_sem, inc=1,
                device_id=(nbr,), device_id_type=pl.DeviceIdType.MESH,
            )
        pl.semaphore_wait(barrier_sem, 2)

        scale = sx_ref[0] * sw_ref[0]

        fwd_ref[0] = x_ref[:m_half, :].astype(jnp.float8_e5m2)
        bwd_ref[0] = x_ref[m_half:, :].astype(jnp.float8_e5m2)

        def make_copy(buf, sems_s, sems_r, h, s, dst):
            rows = pl.ds(s * seg, seg)
            return pltpu.make_async_remote_copy(
                src_ref=buf.at[h, rows],
                dst_ref=buf.at[h + 1, rows],
                send_sem=sems_s.at[h, s],
                recv_sem=sems_r.at[h, s],
                device_id=(dst,),
                device_id_type=pl.DeviceIdType.MESH,
            )

        def gemm_slot(s):
            o_f = lax.rem(my - s + N_DEV, N_DEV)
            acc = jnp.dot(fwd_ref[s].astype(jnp.bfloat16), w_bf_ref[...],
                          preferred_element_type=jnp.float32)
            out_ref[pl.ds(o_f * m_per, m_half), :] = acc * scale
            o_b = lax.rem(my + s, N_DEV)
            acc = jnp.dot(bwd_ref[s].astype(jnp.bfloat16), w_bf_ref[...],
                          preferred_element_type=jnp.float32)
            out_ref[pl.ds(o_b * m_per + m_half, m_half), :] = acc * scale

        for s in range(n_seg):
            make_copy(fwd_ref, fs_sems, fr_sems, 0, s, right).start()
            make_copy(bwd_ref, bs_sems, br_sems, 0, s, left).start()

        w_bf_ref[...] = w_ref[...].astype(jnp.bfloat16)
        gemm_slot(0)

        for h in range(1, N_DEV - 1):
            for s in range(n_seg):
                make_copy(fwd_ref, fs_sems, fr_sems, h - 1, s, right).wait_recv()
                make_copy(fwd_ref, fs_sems, fr_sems, h, s, right).start()
                make_copy(bwd_ref, bs_sems, br_sems, h - 1, s, left).wait_recv()
                make_copy(bwd_ref, bs_sems, br_sems, h, s, left).start()
            gemm_slot(h)
        for s in range(n_seg):
            make_copy(fwd_ref, fs_sems, fr_sems, N_DEV - 2, s, right).wait_recv()
            make_copy(bwd_ref, bs_sems, br_sems, N_DEV - 2, s, left).wait_recv()
        gemm_slot(N_DEV - 1)

        for h in range(N_DEV - 1):
            for s in range(n_seg):
                make_copy(fwd_ref, fs_sems, fr_sems, h, s, right).wait_send()
                make_copy(bwd_ref, bs_sems, br_sems, h, s, left).wait_send()

        @functools.partial(
            pl.run_scoped, second_barrier=pltpu.SemaphoreType.REGULAR
        )
        def _(second_barrier):
            for nbr in (left, right):
                pl.semaphore_signal(
                    second_barrier, inc=1,
                    device_id=(nbr,), device_id_type=pl.DeviceIdType.MESH,
                )
            pl.semaphore_wait(second_barrier, 2)

    return pl.pallas_call(
        body,
        out_shape=jax.ShapeDtypeStruct((m_total, n_per), jnp.float32),
        in_specs=[
            pl.BlockSpec(memory_space=pltpu.VMEM),
            pl.BlockSpec(memory_space=pltpu.VMEM),
            pl.BlockSpec(memory_space=pltpu.SMEM),
            pl.BlockSpec(memory_space=pltpu.SMEM),
        ],
        out_specs=pl.BlockSpec(memory_space=pltpu.VMEM),
        scratch_shapes=[
            pltpu.VMEM((N_DEV, m_half, k), jnp.float8_e5m2),
            pltpu.VMEM((N_DEV, m_half, k), jnp.float8_e5m2),
            pltpu.VMEM((k, n_per), jnp.bfloat16),
            pltpu.SemaphoreType.DMA((N_DEV - 1, 2)),
            pltpu.SemaphoreType.DMA((N_DEV - 1, 2)),
            pltpu.SemaphoreType.DMA((N_DEV - 1, 2)),
            pltpu.SemaphoreType.DMA((N_DEV - 1, 2)),
        ],
        compiler_params=pltpu.CompilerParams(collective_id=0),
    )(x, w_mat, scale_x, scale_w)


# device time: 94915 ns/iter; 1.9538x vs baseline; 1.0109x over previous
import functools

import jax
import jax.numpy as jnp
from jax import lax
from jax.experimental import pallas as pl
from jax.experimental.pallas import tpu as pltpu

N_DEV = 8


def kernel(x, w_mat, scale_x, scale_w):
    m_per, k = x.shape
    n_per = w_mat.shape[1]
    m_total = N_DEV * m_per
    m_half = m_per // 2

    n_seg = 2
    seg = m_half // n_seg

    def body(x_ref, w_ref, sx_ref, sw_ref, out_ref, fwd_ref, bwd_ref,
             w_bf_ref, fs_sems, fr_sems, bs_sems, br_sems):
        my = lax.axis_index("i")
        left = lax.rem(my - 1 + N_DEV, N_DEV)
        right = lax.rem(my + 1, N_DEV)

        barrier_sem = pltpu.get_barrier_semaphore()
        for nbr in (left, right):
            pl.semaphore_signal(
                barrier_sem, inc=1,
                device_id=(nbr,), device_id_type=pl.DeviceIdType.MESH,
            )

        scale = sx_ref[0] * sw_ref[0]

        fwd_ref[0, :seg] = x_ref[:seg, :].astype(jnp.float8_e5m2)
        bwd_ref[0, :seg] = x_ref[m_half:m_half + seg, :].astype(jnp.float8_e5m2)

        pl.semaphore_wait(barrier_sem, 2)

        def make_copy(buf, sems_s, sems_r, h, s, dst):
            rows = pl.ds(s * seg, seg)
            return pltpu.make_async_remote_copy(
                src_ref=buf.at[h, rows],
                dst_ref=buf.at[h + 1, rows],
                send_sem=sems_s.at[h, s],
                recv_sem=sems_r.at[h, s],
                device_id=(dst,),
                device_id_type=pl.DeviceIdType.MESH,
            )

        def gemm_slot(s):
            o_f = lax.rem(my - s + N_DEV, N_DEV)
            acc = jnp.dot(fwd_ref[s].astype(jnp.bfloat16), w_bf_ref[...],
                          preferred_element_type=jnp.float32)
            out_ref[pl.ds(o_f * m_per, m_half), :] = acc * scale
            o_b = lax.rem(my + s, N_DEV)
            acc = jnp.dot(bwd_ref[s].astype(jnp.bfloat16), w_bf_ref[...],
                          preferred_element_type=jnp.float32)
            out_ref[pl.ds(o_b * m_per + m_half, m_half), :] = acc * scale

        make_copy(fwd_ref, fs_sems, fr_sems, 0, 0, right).start()
        make_copy(bwd_ref, bs_sems, br_sems, 0, 0, left).start()
        fwd_ref[0, seg:] = x_ref[seg:m_half, :].astype(jnp.float8_e5m2)
        bwd_ref[0, seg:] = x_ref[m_half + seg:, :].astype(jnp.float8_e5m2)
        make_copy(fwd_ref, fs_sems, fr_sems, 0, 1, right).start()
        make_copy(bwd_ref, bs_sems, br_sems, 0, 1, left).start()

        w_bf_ref[...] = w_ref[...].astype(jnp.bfloat16)
        gemm_slot(0)

        for h in range(1, N_DEV - 1):
            for s in range(n_seg):
                make_copy(fwd_ref, fs_sems, fr_sems, h - 1, s, right).wait_recv()
                make_copy(fwd_ref, fs_sems, fr_sems, h, s, right).start()
                make_copy(bwd_ref, bs_sems, br_sems, h - 1, s, left).wait_recv()
                make_copy(bwd_ref, bs_sems, br_sems, h, s, left).start()
            gemm_slot(h)
        def gemm_strip(buf, slot, s, origin, half_off):
            acc = jnp.dot(
                buf[slot, pl.ds(s * seg, seg)].astype(jnp.bfloat16),
                w_bf_ref[...], preferred_element_type=jnp.float32)
            out_ref[pl.ds(origin * m_per + half_off + s * seg, seg), :] = (
                acc * scale)

        o_f = lax.rem(my + 1, N_DEV)
        o_b = lax.rem(my + 7, N_DEV)
        for s in range(n_seg):
            make_copy(fwd_ref, fs_sems, fr_sems, N_DEV - 2, s, right).wait_recv()
            make_copy(bwd_ref, bs_sems, br_sems, N_DEV - 2, s, left).wait_recv()
            gemm_strip(fwd_ref, N_DEV - 1, s, o_f, 0)
            gemm_strip(bwd_ref, N_DEV - 1, s, o_b, m_half)

        for h in range(N_DEV - 1):
            for s in range(n_seg):
                make_copy(fwd_ref, fs_sems, fr_sems, h, s, right).wait_send()
                make_copy(bwd_ref, bs_sems, br_sems, h, s, left).wait_send()

        @functools.partial(
            pl.run_scoped, second_barrier=pltpu.SemaphoreType.REGULAR
        )
        def _(second_barrier):
            for nbr in (left, right):
                pl.semaphore_signal(
                    second_barrier, inc=1,
                    device_id=(nbr,), device_id_type=pl.DeviceIdType.MESH,
                )
            pl.semaphore_wait(second_barrier, 2)

    return pl.pallas_call(
        body,
        out_shape=jax.ShapeDtypeStruct((m_total, n_per), jnp.float32),
        in_specs=[
            pl.BlockSpec(memory_space=pltpu.VMEM),
            pl.BlockSpec(memory_space=pltpu.VMEM),
            pl.BlockSpec(memory_space=pltpu.SMEM),
            pl.BlockSpec(memory_space=pltpu.SMEM),
        ],
        out_specs=pl.BlockSpec(memory_space=pltpu.VMEM),
        scratch_shapes=[
            pltpu.VMEM((N_DEV, m_half, k), jnp.float8_e5m2),
            pltpu.VMEM((N_DEV, m_half, k), jnp.float8_e5m2),
            pltpu.VMEM((k, n_per), jnp.bfloat16),
            pltpu.SemaphoreType.DMA((N_DEV - 1, 2)),
            pltpu.SemaphoreType.DMA((N_DEV - 1, 2)),
            pltpu.SemaphoreType.DMA((N_DEV - 1, 2)),
            pltpu.SemaphoreType.DMA((N_DEV - 1, 2)),
        ],
        compiler_params=pltpu.CompilerParams(collective_id=0),
    )(x, w_mat, scale_x, scale_w)


# device time: 78772 ns/iter; 2.3542x vs baseline; 1.2049x over previous
import functools

import jax
import jax.numpy as jnp
from jax import lax
from jax.experimental import pallas as pl
from jax.experimental.pallas import tpu as pltpu

N_DEV = 8

ORDER_ROWS = ((0, 128), (128, 192), (320, 192))


def kernel(x, w_mat, scale_x, scale_w):
    m_per, k = x.shape
    n_per = w_mat.shape[1]
    m_total = N_DEV * m_per

    def body(x_ref, w_ref, sx_ref, sw_ref, out_ref, buf_a, buf_b, buf_c,
             w_bf_ref, send_sems, recv_sems):
        my = lax.axis_index("i")

        def flip_x(p):
            q = lax.rem(p, 4)
            return p - q + (q + 1 - 2 * lax.rem(q, 2))

        def flip_y(p):
            q = lax.rem(p, 4)
            return p - q + (3 - q)

        def flip_z(p):
            return lax.rem(p + 4, N_DEV)

        FLIPS = (
            (flip_x, flip_y, flip_z),
            (flip_y, flip_z, flip_x),
            (flip_z, flip_x, flip_y),
        )
        bufs = (buf_a, buf_b, buf_c)

        partners = (flip_x(my), flip_y(my), flip_z(my))
        barrier_sem = pltpu.get_barrier_semaphore()
        for nbr in partners:
            pl.semaphore_signal(
                barrier_sem, inc=1,
                device_id=(nbr,), device_id_type=pl.DeviceIdType.MESH,
            )

        scale = sx_ref[0] * sw_ref[0]

        for o, (off, rows) in enumerate(ORDER_ROWS):
            bufs[o][0] = x_ref[off:off + rows, :].astype(jnp.float8_e5m2)

        pl.semaphore_wait(barrier_sem, 3)

        origins = [[my] for _ in range(3)]

        def copy(o, j, src_slot, dst_slot, partner):
            return pltpu.make_async_remote_copy(
                src_ref=bufs[o].at[src_slot],
                dst_ref=bufs[o].at[dst_slot],
                send_sem=send_sems.at[o, j],
                recv_sem=recv_sems.at[o, j],
                device_id=(partner,),
                device_id_type=pl.DeviceIdType.MESH,
            )

        def gemm_piece(o, slot):
            off, rows = ORDER_ROWS[o]
            acc = jnp.dot(bufs[o][slot].astype(jnp.bfloat16), w_bf_ref[...],
                          preferred_element_type=jnp.float32)
            out_ref[pl.ds(origins[o][slot] * m_per + off, rows), :] = (
                acc * scale)

        K0 = (0, 1, 3)

        def issue_phase(p):
            n = 1 << p
            for o in range(3):
                partner = FLIPS[o][p](my)
                for j in range(n):
                    copy(o, K0[p] + j, j, n + j, partner).start()
                origins[o].extend(FLIPS[o][p](s) for s in origins[o][:n])

        def wait_phase(p):
            n = 1 << p
            for o in range(3):
                for j in range(n):
                    copy(o, K0[p] + j, j, n + j, 0).wait_recv()

        issue_phase(0)
        w_bf_ref[...] = w_ref[...].astype(jnp.bfloat16)
        for o in range(3):
            gemm_piece(o, 0)
        wait_phase(0)

        issue_phase(1)
        for o in range(3):
            gemm_piece(o, 1)
        wait_phase(1)

        issue_phase(2)
        for o in range(3):
            gemm_piece(o, 2)
            gemm_piece(o, 3)
        for j in range(4):
            for o in range(3):
                copy(o, 3 + j, j, 4 + j, 0).wait_recv()
                gemm_piece(o, 4 + j)

        for p in range(3):
            for o in range(3):
                for j in range(1 << p):
                    copy(o, K0[p] + j, j, (1 << p) + j, 0).wait_send()

        @functools.partial(
            pl.run_scoped, second_barrier=pltpu.SemaphoreType.REGULAR
        )
        def _(second_barrier):
            for nbr in partners:
                pl.semaphore_signal(
                    second_barrier, inc=1,
                    device_id=(nbr,), device_id_type=pl.DeviceIdType.MESH,
                )
            pl.semaphore_wait(second_barrier, 3)

    return pl.pallas_call(
        body,
        out_shape=jax.ShapeDtypeStruct((m_total, n_per), jnp.float32),
        in_specs=[
            pl.BlockSpec(memory_space=pltpu.VMEM),
            pl.BlockSpec(memory_space=pltpu.VMEM),
            pl.BlockSpec(memory_space=pltpu.SMEM),
            pl.BlockSpec(memory_space=pltpu.SMEM),
        ],
        out_specs=pl.BlockSpec(memory_space=pltpu.VMEM),
        scratch_shapes=[
            pltpu.VMEM((N_DEV, 128, k), jnp.float8_e5m2),
            pltpu.VMEM((N_DEV, 192, k), jnp.float8_e5m2),
            pltpu.VMEM((N_DEV, 192, k), jnp.float8_e5m2),
            pltpu.VMEM((k, n_per), jnp.bfloat16),
            pltpu.SemaphoreType.DMA((3, 7)),
            pltpu.SemaphoreType.DMA((3, 7)),
        ],
        compiler_params=pltpu.CompilerParams(collective_id=0),
    )(x, w_mat, scale_x, scale_w)


# device time: 72528 ns/iter; 2.5569x vs baseline; 1.0861x over previous
import functools

import jax
import jax.numpy as jnp
from jax import lax
from jax.experimental import pallas as pl
from jax.experimental.pallas import tpu as pltpu

N_DEV = 8

ORDER_ROWS = ((0, 128), (128, 192), (320, 192))


def kernel(x, w_mat, scale_x, scale_w):
    m_per, k = x.shape
    n_per = w_mat.shape[1]
    m_total = N_DEV * m_per

    def body(x_ref, w_ref, sx_ref, sw_ref, out_ref, buf_a, buf_b, buf_c,
             w_bf_ref, send_sems, recv_sems):
        my = lax.axis_index("i")

        def flip_x(p):
            q = lax.rem(p, 4)
            return p - q + (q + 1 - 2 * lax.rem(q, 2))

        def flip_y(p):
            q = lax.rem(p, 4)
            return p - q + (3 - q)

        def flip_z(p):
            return lax.rem(p + 4, N_DEV)

        FLIPS = (
            (flip_x, flip_y, flip_z),
            (flip_y, flip_z, flip_x),
            (flip_z, flip_x, flip_y),
        )
        bufs = (buf_a, buf_b, buf_c)

        partners = (flip_x(my), flip_y(my), flip_z(my))
        barrier_sem = pltpu.get_barrier_semaphore()
        for nbr in partners:
            pl.semaphore_signal(
                barrier_sem, inc=1,
                device_id=(nbr,), device_id_type=pl.DeviceIdType.MESH,
            )

        scale = sx_ref[0] * sw_ref[0]

        for o, (off, rows) in enumerate(ORDER_ROWS):
            bufs[o][0] = x_ref[off:off + rows, :].astype(jnp.float8_e5m2)

        pl.semaphore_wait(barrier_sem, 3)

        origins = [[my] for _ in range(3)]

        def copy(o, j, src_slot, dst_slot, partner):
            return pltpu.make_async_remote_copy(
                src_ref=bufs[o].at[src_slot],
                dst_ref=bufs[o].at[dst_slot],
                send_sem=send_sems.at[o, j],
                recv_sem=recv_sems.at[o, j],
                device_id=(partner,),
                device_id_type=pl.DeviceIdType.MESH,
            )

        def gemm_piece(o, slot):
            off, rows = ORDER_ROWS[o]
            acc = jnp.dot(bufs[o][slot].astype(jnp.bfloat16), w_bf_ref[...],
                          preferred_element_type=jnp.float32)
            out_ref[pl.ds(origins[o][slot] * m_per + off, rows), :] = (
                acc * scale)

        K0 = (0, 1, 3)

        for p in range(3):
            n = 1 << p
            for o in range(3):
                origins[o].extend(FLIPS[o][p](s) for s in origins[o][:n])

        def send(o, p, j, src_slot, dst_slot):
            copy(o, K0[p] + j, src_slot, dst_slot,
                 FLIPS[o][p](my)).start()

        def recv(o, p, j, src_slot, dst_slot):
            copy(o, K0[p] + j, src_slot, dst_slot, 0).wait_recv()

        for o in range(3):
            send(o, 0, 0, 0, 1)
            send(o, 2, 0, 0, 4)
        w_bf_ref[...] = w_ref[...].astype(jnp.bfloat16)
        for o in range(3):
            gemm_piece(o, 0)

        for o in range(3):
            recv(o, 0, 0, 0, 1)
            send(o, 1, 0, 0, 2)
            send(o, 1, 1, 1, 3)
            send(o, 2, 1, 1, 5)
        for o in range(3):
            gemm_piece(o, 1)

        for o in range(3):
            recv(o, 1, 0, 0, 2)
            send(o, 2, 2, 2, 6)
            recv(o, 1, 1, 1, 3)
            send(o, 2, 3, 3, 7)
        for o in range(3):
            gemm_piece(o, 2)
            gemm_piece(o, 3)
        for j in range(4):
            for o in range(3):
                recv(o, 2, j, j, 4 + j)
                gemm_piece(o, 4 + j)

        for p in range(3):
            for o in range(3):
                for j in range(1 << p):
                    copy(o, K0[p] + j, j, (1 << p) + j, 0).wait_send()

        @functools.partial(
            pl.run_scoped, second_barrier=pltpu.SemaphoreType.REGULAR
        )
        def _(second_barrier):
            for nbr in partners:
                pl.semaphore_signal(
                    second_barrier, inc=1,
                    device_id=(nbr,), device_id_type=pl.DeviceIdType.MESH,
                )
            pl.semaphore_wait(second_barrier, 3)

    return pl.pallas_call(
        body,
        out_shape=jax.ShapeDtypeStruct((m_total, n_per), jnp.float32),
        in_specs=[
            pl.BlockSpec(memory_space=pltpu.VMEM),
            pl.BlockSpec(memory_space=pltpu.VMEM),
            pl.BlockSpec(memory_space=pltpu.SMEM),
            pl.BlockSpec(memory_space=pltpu.SMEM),
        ],
        out_specs=pl.BlockSpec(memory_space=pltpu.VMEM),
        scratch_shapes=[
            pltpu.VMEM((N_DEV, 128, k), jnp.float8_e5m2),
            pltpu.VMEM((N_DEV, 192, k), jnp.float8_e5m2),
            pltpu.VMEM((N_DEV, 192, k), jnp.float8_e5m2),
            pltpu.VMEM((k, n_per), jnp.bfloat16),
            pltpu.SemaphoreType.DMA((3, 7)),
            pltpu.SemaphoreType.DMA((3, 7)),
        ],
        compiler_params=pltpu.CompilerParams(collective_id=0),
    )(x, w_mat, scale_x, scale_w)
